# Initial kernel scaffold; baseline (speedup 1.0000x reference)
import jax
import jax.numpy as jnp
from jax.experimental import pallas as pl


def kernel(input_embeddings, target_words, out_emb_weight):
    raise NotImplementedError("write your pallas kernel here")


# speedup vs baseline: 3.7461x; 3.7461x over previous
import functools

import jax
import jax.numpy as jnp
import numpy as np
from jax import lax
from jax.experimental import pallas as pl
from jax.experimental.pallas import tpu as pltpu
from jax.experimental.pallas import tpu_sc as plsc

BATCH = 16384
DIM = 64
VOCAB = 100000
NOISE_VOCAB = 64
NUM_NEG = 5

NUM_CORES = 2
NUM_SUBCORES = 16
NUM_WORKERS = NUM_CORES * NUM_SUBCORES
B_PER_W = BATCH // NUM_WORKERS
CHUNK = 128
NCHUNK = B_PER_W // CHUNK

TC_BLOCK = 512
TC_GRID = BATCH // TC_BLOCK


def _noise_constants():
    probs = jnp.full((NOISE_VOCAB,), 1.0 / NOISE_VOCAB, dtype=jnp.float32)
    logits = jnp.log(probs)
    skey = jax.random.key(42)
    s1, s2 = jax.random.split(skey)
    neg = np.asarray(jax.random.categorical(s1, logits, shape=(BATCH, NUM_NEG)))
    repl = np.asarray(jax.random.categorical(s2, logits, shape=(BATCH, NUM_NEG)))

    def pack(a):
        p = np.zeros((BATCH,), dtype=np.int64)
        for k in range(NUM_NEG):
            p |= a[:, k].astype(np.int64) << (6 * k)
        return p.astype(np.int32).reshape(TC_GRID, 1, TC_BLOCK)

    return pack(neg), pack(repl)


_NEG_PACKED, _REPL_PACKED = _noise_constants()


def _sc_gather_body(table_hbm, idx_hbm, out_hbm, idx_v, rows_v, sem):
    wid = lax.axis_index("s") * NUM_CORES + lax.axis_index("c")
    base = wid * B_PER_W
    for j in range(NCHUNK):
        pltpu.sync_copy(idx_hbm.at[pl.ds(base + j * CHUNK, CHUNK)], idx_v.at[j])
    copies = [
        pltpu.async_copy(table_hbm.at[idx_v.at[j]], rows_v.at[j], sem)
        for j in range(NCHUNK)
    ]
    for c in copies:
        c.wait()
    for j in range(NCHUNK):
        pltpu.sync_copy(rows_v.at[j], out_hbm.at[pl.ds(base + j * CHUNK, CHUNK)])


@functools.lru_cache(maxsize=None)
def _build_sc_gather():
    return functools.partial(
        pl.kernel,
        mesh=plsc.VectorSubcoreMesh(
            core_axis_name="c", subcore_axis_name="s",
            num_cores=NUM_CORES, num_subcores=NUM_SUBCORES,
        ),
        out_type=jax.ShapeDtypeStruct((BATCH, DIM), jnp.float32),
        compiler_params=pltpu.CompilerParams(use_tc_tiling_on_sc=False),
        scratch_types=[
            pltpu.VMEM((NCHUNK, CHUNK), jnp.int32),
            pltpu.VMEM((NCHUNK, CHUNK, DIM), jnp.float32),
            pltpu.SemaphoreType.DMA,
        ],
    )(_sc_gather_body)


def _log_sigmoid(z):
    return jnp.minimum(z, 0.0) - jnp.log1p(jnp.exp(-jnp.abs(z)))


def _tc_loss_body(x_ref, pos_ref, w_ref, tgt_ref, negp_ref, replp_ref, out_ref):
    i = pl.program_id(0)

    x = x_ref[...]
    p = pos_ref[...]
    w = w_ref[...]
    tgt = tgt_ref[0]
    negp = negp_ref[0]
    replp = replp_ref[0]

    s_t = lax.dot_general(
        w, x, dimension_numbers=(((1,), (1,)), ((), ())),
        preferred_element_type=jnp.float32,
        precision=lax.Precision.HIGHEST,
    )

    iota_v = lax.broadcasted_iota(jnp.int32, (NOISE_VOCAB, TC_BLOCK), 0)
    total = jnp.float32(0.0)
    for k in range(NUM_NEG):
        nk = (negp >> (6 * k)) & 63
        rk = (replp >> (6 * k)) & 63
        nwk = jnp.where(nk == tgt, rk, nk)
        sel = jnp.where(iota_v == nwk, s_t, 0.0)
        sk = jnp.sum(sel, axis=0)
        total += jnp.sum(_log_sigmoid(-sk))

    pos_score = jnp.sum(x * p, axis=1)
    total += jnp.sum(_log_sigmoid(pos_score))

    @pl.when(i == 0)
    def _init():
        out_ref[...] = jnp.zeros((1, 1), jnp.float32)

    out_ref[...] += jnp.full((1, 1), total, jnp.float32)

    @pl.when(i == TC_GRID - 1)
    def _fin():
        out_ref[...] = out_ref[...] * (-1.0 / BATCH)


_tc_loss = pl.pallas_call(
    _tc_loss_body,
    grid=(TC_GRID,),
    in_specs=[
        pl.BlockSpec((TC_BLOCK, DIM), lambda i: (i, 0)),
        pl.BlockSpec((TC_BLOCK, DIM), lambda i: (i, 0)),
        pl.BlockSpec((NOISE_VOCAB, DIM), lambda i: (0, 0)),
        pl.BlockSpec((1, 1, TC_BLOCK), lambda i: (i, 0, 0)),
        pl.BlockSpec((1, 1, TC_BLOCK), lambda i: (i, 0, 0)),
        pl.BlockSpec((1, 1, TC_BLOCK), lambda i: (i, 0, 0)),
    ],
    out_specs=pl.BlockSpec((1, 1), lambda i: (0, 0)),
    out_shape=jax.ShapeDtypeStruct((1, 1), jnp.float32),
)


def kernel(input_embeddings, target_words, out_emb_weight):
    pos_emb = _build_sc_gather()(out_emb_weight, target_words)
    w64 = out_emb_weight[:NOISE_VOCAB]
    tgt3 = target_words.reshape(TC_GRID, 1, TC_BLOCK)
    res = _tc_loss(
        input_embeddings, pos_emb, w64, tgt3,
        jnp.asarray(_NEG_PACKED), jnp.asarray(_REPL_PACKED),
    )
    return res.reshape(())


# speedup vs baseline: 4.2349x; 1.1305x over previous
import functools

import jax
import jax.numpy as jnp
import numpy as np
from jax import lax
from jax.experimental import pallas as pl
from jax.experimental.pallas import tpu as pltpu
from jax.experimental.pallas import tpu_sc as plsc

BATCH = 16384
DIM = 64
VOCAB = 100000
NOISE_VOCAB = 64
NUM_NEG = 5

NUM_CORES = 2
NUM_SUBCORES = 16
NUM_WORKERS = NUM_CORES * NUM_SUBCORES
D_PER_W = DIM // NUM_WORKERS
E_CHUNK = 2048
N_ECHUNK = BATCH // E_CHUNK
LANES = 16

TC_BLOCK = 512
TC_GRID = BATCH // TC_BLOCK


def _threefry2x32(k1, k2, x0, x1):
    k1 = np.uint32(k1)
    k2 = np.uint32(k2)
    ks = (k1, k2, k1 ^ k2 ^ np.uint32(0x1BD11BDA))
    x0 = (x0 + ks[0]).astype(np.uint32)
    x1 = (x1 + ks[1]).astype(np.uint32)

    def rounds(x0, x1, rots):
        for r in rots:
            x0 = (x0 + x1).astype(np.uint32)
            x1 = (x1 << np.uint32(r)) | (x1 >> np.uint32(32 - r))
            x1 = x0 ^ x1
        return x0, x1

    rot_a, rot_b = (13, 15, 26, 6), (17, 29, 16, 24)
    inject = ((ks[1], ks[2], 1), (ks[2], ks[0], 2), (ks[0], ks[1], 3),
              (ks[1], ks[2], 4), (ks[2], ks[0], 5))
    for (a, b, c), rt in zip(inject, (rot_a, rot_b, rot_a, rot_b, rot_a)):
        x0, x1 = rounds(x0, x1, rt)
        x0 = (x0 + a).astype(np.uint32)
        x1 = (x1 + b + np.uint32(c)).astype(np.uint32)
    return x0, x1


def _iota_pair(size):
    n = np.arange(size, dtype=np.uint64)
    return ((n >> np.uint64(32)).astype(np.uint32),
            (n & np.uint64(0xFFFFFFFF)).astype(np.uint32))


def _categorical_uniform(k1, k2):
    hi, lo = _iota_pair(BATCH * NUM_NEG * NOISE_VOCAB)
    b1, b2 = _threefry2x32(k1, k2, hi, lo)
    bits = b1 ^ b2
    fb = (bits >> np.uint32(9)) | np.uint32(0x3F800000)
    floats = fb.view(np.float32) - np.float32(1.0)
    tiny = np.float32(np.finfo(np.float32).tiny)
    u = np.maximum(tiny, floats * np.float32(np.float32(1.0) - tiny) + tiny)
    g = -np.log(-np.log(u))
    return np.argmax(g.reshape(BATCH, NUM_NEG, NOISE_VOCAB), axis=-1).astype(np.int32)


def _noise_constants():
    hi, lo = _iota_pair(2)
    b1, b2 = _threefry2x32(0, 42, hi, lo)
    neg = _categorical_uniform(b1[0], b2[0])
    repl = _categorical_uniform(b1[1], b2[1])

    def pack(a):
        p = np.zeros((BATCH,), dtype=np.int64)
        for k in range(NUM_NEG):
            p |= a[:, k].astype(np.int64) << (6 * k)
        return p.astype(np.int32).reshape(TC_GRID, 1, TC_BLOCK)

    return pack(neg), pack(repl)


_NEG_PACKED, _REPL_PACKED = _noise_constants()


def _sc_gather_body(wt_hbm, tgt_hbm, out_hbm, tgt_v, row_v, g_v):
    wid = lax.axis_index("s") * NUM_CORES + lax.axis_index("c")
    pltpu.sync_copy(tgt_hbm, tgt_v)
    for rep in range(D_PER_W):
        d = wid * D_PER_W + rep
        pltpu.sync_copy(wt_hbm.at[d], row_v)

        for c in range(N_ECHUNK):
            def body(i, carry, c=c):
                base = c * E_CHUNK + i * LANES
                idx16 = tgt_v[pl.ds(base, LANES)]
                g_v[pl.ds(i * LANES, LANES)] = plsc.load_gather(row_v, [idx16])
                return carry

            lax.fori_loop(0, E_CHUNK // LANES, body, 0)
            pltpu.sync_copy(g_v, out_hbm.at[d, pl.ds(c * E_CHUNK, E_CHUNK)])


@functools.lru_cache(maxsize=None)
def _build_sc_gather():
    return functools.partial(
        pl.kernel,
        mesh=plsc.VectorSubcoreMesh(
            core_axis_name="c", subcore_axis_name="s",
            num_cores=NUM_CORES, num_subcores=NUM_SUBCORES,
        ),
        out_type=jax.ShapeDtypeStruct((DIM, BATCH), jnp.float32),
        compiler_params=pltpu.CompilerParams(
            use_tc_tiling_on_sc=False, needs_layout_passes=False),
        scratch_types=[
            pltpu.VMEM((BATCH,), jnp.int32),
            pltpu.VMEM((VOCAB,), jnp.float32),
            pltpu.VMEM((E_CHUNK,), jnp.float32),
        ],
    )(_sc_gather_body)


def _log_sigmoid(z):
    return jnp.minimum(z, 0.0) - jnp.log1p(jnp.exp(-jnp.abs(z)))


def _tc_loss_body(x_ref, g_ref, w_ref, tgt_ref, negp_ref, replp_ref, out_ref):
    i = pl.program_id(0)

    x = x_ref[...]
    g = g_ref[...]
    w = w_ref[...]
    tgt = tgt_ref[0]
    negp = negp_ref[0]
    replp = replp_ref[0]

    s_t = lax.dot_general(
        w, x, dimension_numbers=(((0,), (0,)), ((), ())),
        preferred_element_type=jnp.float32,
    )

    iota_v = lax.broadcasted_iota(jnp.int32, (NOISE_VOCAB, TC_BLOCK), 0)
    total = jnp.float32(0.0)
    for k in range(NUM_NEG):
        nk = (negp >> (6 * k)) & 63
        rk = (replp >> (6 * k)) & 63
        nwk = jnp.where(nk == tgt, rk, nk)
        sel = jnp.where(iota_v == nwk, s_t, 0.0)
        sk = jnp.sum(sel, axis=0)
        total += jnp.sum(_log_sigmoid(-sk))

    pos_score = jnp.sum(x * g, axis=0)
    total += jnp.sum(_log_sigmoid(pos_score))

    @pl.when(i == 0)
    def _init():
        out_ref[...] = jnp.zeros((1, 1), jnp.float32)

    out_ref[...] += jnp.full((1, 1), total, jnp.float32)

    @pl.when(i == TC_GRID - 1)
    def _fin():
        out_ref[...] = out_ref[...] * (-1.0 / BATCH)


_tc_loss = pl.pallas_call(
    _tc_loss_body,
    grid=(TC_GRID,),
    in_specs=[
        pl.BlockSpec((DIM, TC_BLOCK), lambda i: (0, i)),
        pl.BlockSpec((DIM, TC_BLOCK), lambda i: (0, i)),
        pl.BlockSpec((DIM, NOISE_VOCAB), lambda i: (0, 0)),
        pl.BlockSpec((1, 1, TC_BLOCK), lambda i: (i, 0, 0)),
        pl.BlockSpec((1, 1, TC_BLOCK), lambda i: (i, 0, 0)),
        pl.BlockSpec((1, 1, TC_BLOCK), lambda i: (i, 0, 0)),
    ],
    out_specs=pl.BlockSpec((1, 1), lambda i: (0, 0)),
    out_shape=jax.ShapeDtypeStruct((1, 1), jnp.float32),
)


def kernel(input_embeddings, target_words, out_emb_weight):
    w_t = out_emb_weight.T
    x_t = input_embeddings.T
    g = _build_sc_gather()(w_t, target_words)
    w64_t = lax.slice(w_t, (0, 0), (DIM, NOISE_VOCAB))
    tgt3 = target_words.reshape(TC_GRID, 1, TC_BLOCK)
    res = _tc_loss(
        x_t, g, w64_t, tgt3,
        jnp.asarray(_NEG_PACKED), jnp.asarray(_REPL_PACKED),
    )
    return res.reshape(())


# speedup vs baseline: 6.0084x; 1.4188x over previous
import functools

import jax
import jax.numpy as jnp
import numpy as np
from jax import lax
from jax.experimental import pallas as pl
from jax.experimental.pallas import tpu as pltpu
from jax.experimental.pallas import tpu_sc as plsc

BATCH = 16384
DIM = 64
VOCAB = 100000
NOISE_VOCAB = 64
NUM_NEG = 5

NUM_CORES = 2
NUM_SUBCORES = 16
NUM_WORKERS = NUM_CORES * NUM_SUBCORES
D_PER_W = DIM // NUM_WORKERS
E_CHUNK = 2048
N_ECHUNK = BATCH // E_CHUNK
LANES = 16

TC_BLOCK = 512
TC_GRID = BATCH // TC_BLOCK


def _threefry2x32(k1, k2, x0, x1):
    k1 = np.uint32(k1)
    k2 = np.uint32(k2)
    ks = (k1, k2, k1 ^ k2 ^ np.uint32(0x1BD11BDA))
    x0 = (x0 + ks[0]).astype(np.uint32)
    x1 = (x1 + ks[1]).astype(np.uint32)

    def rounds(x0, x1, rots):
        for r in rots:
            x0 = (x0 + x1).astype(np.uint32)
            x1 = (x1 << np.uint32(r)) | (x1 >> np.uint32(32 - r))
            x1 = x0 ^ x1
        return x0, x1

    rot_a, rot_b = (13, 15, 26, 6), (17, 29, 16, 24)
    inject = ((ks[1], ks[2], 1), (ks[2], ks[0], 2), (ks[0], ks[1], 3),
              (ks[1], ks[2], 4), (ks[2], ks[0], 5))
    for (a, b, c), rt in zip(inject, (rot_a, rot_b, rot_a, rot_b, rot_a)):
        x0, x1 = rounds(x0, x1, rt)
        x0 = (x0 + a).astype(np.uint32)
        x1 = (x1 + b + np.uint32(c)).astype(np.uint32)
    return x0, x1


def _iota_pair(size):
    n = np.arange(size, dtype=np.uint64)
    return ((n >> np.uint64(32)).astype(np.uint32),
            (n & np.uint64(0xFFFFFFFF)).astype(np.uint32))


def _categorical_uniform(k1, k2):
    hi, lo = _iota_pair(BATCH * NUM_NEG * NOISE_VOCAB)
    b1, b2 = _threefry2x32(k1, k2, hi, lo)
    bits = b1 ^ b2
    fb = (bits >> np.uint32(9)) | np.uint32(0x3F800000)
    floats = fb.view(np.float32) - np.float32(1.0)
    tiny = np.float32(np.finfo(np.float32).tiny)
    u = np.maximum(tiny, floats * np.float32(np.float32(1.0) - tiny) + tiny)
    g = -np.log(-np.log(u))
    return np.argmax(g.reshape(BATCH, NUM_NEG, NOISE_VOCAB), axis=-1).astype(np.int32)


def _noise_constants():
    hi, lo = _iota_pair(2)
    b1, b2 = _threefry2x32(0, 42, hi, lo)
    neg = _categorical_uniform(b1[0], b2[0])
    repl = _categorical_uniform(b1[1], b2[1])

    def pack(a):
        p = np.zeros((BATCH,), dtype=np.int64)
        for k in range(NUM_NEG):
            p |= a[:, k].astype(np.int64) << (6 * k)
        return p.astype(np.int32).reshape(TC_GRID, 1, TC_BLOCK)

    return pack(neg), pack(repl)


_NEG_PACKED, _REPL_PACKED = _noise_constants()


def _sc_gather_body(wt_hbm, tgt_hbm, out_hbm, tgt_v, row_v, g_v):
    wid = lax.axis_index("s") * NUM_CORES + lax.axis_index("c")
    pltpu.sync_copy(tgt_hbm, tgt_v)
    for rep in range(D_PER_W):
        d = wid * D_PER_W + rep
        pltpu.sync_copy(wt_hbm.at[d], row_v)

        for c in range(N_ECHUNK):
            def body(i, carry, c=c):
                base = c * E_CHUNK + i * LANES
                idx16 = tgt_v[pl.ds(base, LANES)]
                g_v[pl.ds(i * LANES, LANES)] = plsc.load_gather(row_v, [idx16])
                return carry

            lax.fori_loop(0, E_CHUNK // LANES, body, 0)
            pltpu.sync_copy(g_v, out_hbm.at[d, pl.ds(c * E_CHUNK, E_CHUNK)])


@functools.lru_cache(maxsize=None)
def _build_sc_gather():
    return functools.partial(
        pl.kernel,
        mesh=plsc.VectorSubcoreMesh(
            core_axis_name="c", subcore_axis_name="s",
            num_cores=NUM_CORES, num_subcores=NUM_SUBCORES,
        ),
        out_type=jax.ShapeDtypeStruct((DIM, BATCH), jnp.float32),
        compiler_params=pltpu.CompilerParams(
            use_tc_tiling_on_sc=True, needs_layout_passes=False),
        scratch_types=[
            pltpu.VMEM((BATCH,), jnp.int32),
            pltpu.VMEM((VOCAB,), jnp.float32),
            pltpu.VMEM((E_CHUNK,), jnp.float32),
        ],
    )(_sc_gather_body)


def _log_sigmoid(z):
    return jnp.minimum(z, 0.0) - jnp.log1p(jnp.exp(-jnp.abs(z)))


def _tc_loss_body(x_ref, g_ref, w_ref, tgt_ref, negp_ref, replp_ref, out_ref):
    i = pl.program_id(0)

    x = x_ref[...]
    g = g_ref[...]
    w = w_ref[...]
    tgt = tgt_ref[0]
    negp = negp_ref[0]
    replp = replp_ref[0]

    s_t = lax.dot_general(
        w, x, dimension_numbers=(((0,), (0,)), ((), ())),
        preferred_element_type=jnp.float32,
    )

    iota_v = lax.broadcasted_iota(jnp.int32, (NOISE_VOCAB, TC_BLOCK), 0)
    total = jnp.float32(0.0)
    for k in range(NUM_NEG):
        nk = (negp >> (6 * k)) & 63
        rk = (replp >> (6 * k)) & 63
        nwk = jnp.where(nk == tgt, rk, nk)
        sel = jnp.where(iota_v == nwk, s_t, 0.0)
        sk = jnp.sum(sel, axis=0)
        total += jnp.sum(_log_sigmoid(-sk))

    pos_score = jnp.sum(x * g, axis=0)
    total += jnp.sum(_log_sigmoid(pos_score))

    @pl.when(i == 0)
    def _init():
        out_ref[...] = jnp.zeros((1, 1), jnp.float32)

    out_ref[...] += jnp.full((1, 1), total, jnp.float32)

    @pl.when(i == TC_GRID - 1)
    def _fin():
        out_ref[...] = out_ref[...] * (-1.0 / BATCH)


_tc_loss = pl.pallas_call(
    _tc_loss_body,
    grid=(TC_GRID,),
    in_specs=[
        pl.BlockSpec((DIM, TC_BLOCK), lambda i: (0, i)),
        pl.BlockSpec((DIM, TC_BLOCK), lambda i: (0, i)),
        pl.BlockSpec((DIM, NOISE_VOCAB), lambda i: (0, 0)),
        pl.BlockSpec((1, 1, TC_BLOCK), lambda i: (i, 0, 0)),
        pl.BlockSpec((1, 1, TC_BLOCK), lambda i: (i, 0, 0)),
        pl.BlockSpec((1, 1, TC_BLOCK), lambda i: (i, 0, 0)),
    ],
    out_specs=pl.BlockSpec((1, 1), lambda i: (0, 0)),
    out_shape=jax.ShapeDtypeStruct((1, 1), jnp.float32),
)


def kernel(input_embeddings, target_words, out_emb_weight):
    w_t = out_emb_weight.T
    x_t = input_embeddings.T
    g = _build_sc_gather()(w_t, target_words)
    w64_t = lax.slice(w_t, (0, 0), (DIM, NOISE_VOCAB))
    tgt3 = target_words.reshape(TC_GRID, 1, TC_BLOCK)
    res = _tc_loss(
        x_t, g, w64_t, tgt3,
        jnp.asarray(_NEG_PACKED), jnp.asarray(_REPL_PACKED),
    )
    return res.reshape(())


# speedup vs baseline: 6.9713x; 1.1603x over previous
import functools

import jax
import jax.numpy as jnp
import numpy as np
from jax import lax
from jax.experimental import pallas as pl
from jax.experimental.pallas import tpu as pltpu
from jax.experimental.pallas import tpu_sc as plsc

BATCH = 16384
DIM = 64
VOCAB = 100000
NOISE_VOCAB = 64
NUM_NEG = 5

NUM_CORES = 2
NUM_SUBCORES = 16
NUM_WORKERS = NUM_CORES * NUM_SUBCORES
D_PER_W = DIM // NUM_WORKERS
E_CHUNK = 4096
N_ECHUNK = BATCH // E_CHUNK
LANES = 16
UNROLL = 8

TC_BLOCK = 512
TC_GRID = BATCH // TC_BLOCK


def _threefry2x32(k1, k2, x0, x1):
    k1 = np.uint32(k1)
    k2 = np.uint32(k2)
    ks = (k1, k2, k1 ^ k2 ^ np.uint32(0x1BD11BDA))
    x0 = (x0 + ks[0]).astype(np.uint32)
    x1 = (x1 + ks[1]).astype(np.uint32)

    def rounds(x0, x1, rots):
        for r in rots:
            x0 = (x0 + x1).astype(np.uint32)
            x1 = (x1 << np.uint32(r)) | (x1 >> np.uint32(32 - r))
            x1 = x0 ^ x1
        return x0, x1

    rot_a, rot_b = (13, 15, 26, 6), (17, 29, 16, 24)
    inject = ((ks[1], ks[2], 1), (ks[2], ks[0], 2), (ks[0], ks[1], 3),
              (ks[1], ks[2], 4), (ks[2], ks[0], 5))
    for (a, b, c), rt in zip(inject, (rot_a, rot_b, rot_a, rot_b, rot_a)):
        x0, x1 = rounds(x0, x1, rt)
        x0 = (x0 + a).astype(np.uint32)
        x1 = (x1 + b + np.uint32(c)).astype(np.uint32)
    return x0, x1


def _iota_pair(size):
    n = np.arange(size, dtype=np.uint64)
    return ((n >> np.uint64(32)).astype(np.uint32),
            (n & np.uint64(0xFFFFFFFF)).astype(np.uint32))


def _categorical_uniform(k1, k2):
    hi, lo = _iota_pair(BATCH * NUM_NEG * NOISE_VOCAB)
    b1, b2 = _threefry2x32(k1, k2, hi, lo)
    bits = b1 ^ b2
    fb = (bits >> np.uint32(9)) | np.uint32(0x3F800000)
    floats = fb.view(np.float32) - np.float32(1.0)
    tiny = np.float32(np.finfo(np.float32).tiny)
    u = np.maximum(tiny, floats * np.float32(np.float32(1.0) - tiny) + tiny)
    g = -np.log(-np.log(u))
    return np.argmax(g.reshape(BATCH, NUM_NEG, NOISE_VOCAB), axis=-1).astype(np.int32)


def _noise_constants():
    hi, lo = _iota_pair(2)
    b1, b2 = _threefry2x32(0, 42, hi, lo)
    neg = _categorical_uniform(b1[0], b2[0])
    repl = _categorical_uniform(b1[1], b2[1])

    def pack(a):
        p = np.zeros((BATCH,), dtype=np.int64)
        for k in range(NUM_NEG):
            p |= a[:, k].astype(np.int64) << (6 * k)
        return p.astype(np.int32).reshape(TC_GRID, 1, TC_BLOCK)

    return pack(neg), pack(repl)


_NEG_PACKED, _REPL_PACKED = _noise_constants()


def _sc_gather_body(wt_hbm, tgt_hbm, out_hbm, tgt_v, row_v, g_v0, g_v1, sem_out):
    wid = lax.axis_index("s") * NUM_CORES + lax.axis_index("c")
    pltpu.sync_copy(tgt_hbm, tgt_v)
    pending = {}
    step = LANES * UNROLL
    for rep in range(D_PER_W):
        d = wid * D_PER_W + rep
        pltpu.sync_copy(wt_hbm.at[d], row_v)

        for c in range(N_ECHUNK):
            slot = c % 2
            if slot in pending:
                pending.pop(slot).wait()
            g_slot = (g_v0, g_v1)[slot]

            def body(i, carry, c=c, g_slot=g_slot):
                base = c * E_CHUNK + i * step
                for u in range(UNROLL):
                    idx16 = tgt_v[pl.ds(base + u * LANES, LANES)]
                    g_slot[pl.ds(i * step + u * LANES, LANES)] = (
                        plsc.load_gather(row_v, [idx16]))
                return carry

            lax.fori_loop(0, E_CHUNK // step, body, 0)
            pending[slot] = pltpu.async_copy(
                g_slot, out_hbm.at[d, pl.ds(c * E_CHUNK, E_CHUNK)],
                sem_out.at[slot])
    for cp in pending.values():
        cp.wait()


@functools.lru_cache(maxsize=None)
def _build_sc_gather():
    return functools.partial(
        pl.kernel,
        mesh=plsc.VectorSubcoreMesh(
            core_axis_name="c", subcore_axis_name="s",
            num_cores=NUM_CORES, num_subcores=NUM_SUBCORES,
        ),
        out_type=jax.ShapeDtypeStruct((DIM, BATCH), jnp.float32),
        compiler_params=pltpu.CompilerParams(
            use_tc_tiling_on_sc=True, needs_layout_passes=False),
        scratch_types=[
            pltpu.VMEM((BATCH,), jnp.int32),
            pltpu.VMEM((VOCAB,), jnp.float32),
            pltpu.VMEM((E_CHUNK,), jnp.float32),
            pltpu.VMEM((E_CHUNK,), jnp.float32),
            pltpu.SemaphoreType.DMA((2,)),
        ],
    )(_sc_gather_body)


def _log_sigmoid(z):
    return jnp.minimum(z, 0.0) - jnp.log1p(jnp.exp(-jnp.abs(z)))


def _tc_loss_body(x_ref, g_ref, w_ref, tgt_ref, negp_ref, replp_ref, out_ref):
    i = pl.program_id(0)

    x = x_ref[...]
    g = g_ref[...]
    w = w_ref[...]
    tgt = tgt_ref[0]
    negp = negp_ref[0]
    replp = replp_ref[0]

    s_t = lax.dot_general(
        w, x, dimension_numbers=(((0,), (0,)), ((), ())),
        preferred_element_type=jnp.float32,
    )

    iota_v = lax.broadcasted_iota(jnp.int32, (NOISE_VOCAB, TC_BLOCK), 0)
    total = jnp.float32(0.0)
    for k in range(NUM_NEG):
        nk = (negp >> (6 * k)) & 63
        rk = (replp >> (6 * k)) & 63
        nwk = jnp.where(nk == tgt, rk, nk)
        sel = jnp.where(iota_v == nwk, s_t, 0.0)
        sk = jnp.sum(sel, axis=0)
        total += jnp.sum(_log_sigmoid(-sk))

    pos_score = jnp.sum(x * g, axis=0)
    total += jnp.sum(_log_sigmoid(pos_score))

    @pl.when(i == 0)
    def _init():
        out_ref[...] = jnp.zeros((1, 1), jnp.float32)

    out_ref[...] += jnp.full((1, 1), total, jnp.float32)

    @pl.when(i == TC_GRID - 1)
    def _fin():
        out_ref[...] = out_ref[...] * (-1.0 / BATCH)


_tc_loss = pl.pallas_call(
    _tc_loss_body,
    grid=(TC_GRID,),
    in_specs=[
        pl.BlockSpec((DIM, TC_BLOCK), lambda i: (0, i)),
        pl.BlockSpec((DIM, TC_BLOCK), lambda i: (0, i)),
        pl.BlockSpec((DIM, NOISE_VOCAB), lambda i: (0, 0)),
        pl.BlockSpec((1, 1, TC_BLOCK), lambda i: (i, 0, 0)),
        pl.BlockSpec((1, 1, TC_BLOCK), lambda i: (i, 0, 0)),
        pl.BlockSpec((1, 1, TC_BLOCK), lambda i: (i, 0, 0)),
    ],
    out_specs=pl.BlockSpec((1, 1), lambda i: (0, 0)),
    out_shape=jax.ShapeDtypeStruct((1, 1), jnp.float32),
)


def kernel(input_embeddings, target_words, out_emb_weight):
    w_t = out_emb_weight.T
    x_t = input_embeddings.T
    g = _build_sc_gather()(w_t, target_words)
    w64_t = lax.slice(w_t, (0, 0), (DIM, NOISE_VOCAB))
    tgt3 = target_words.reshape(TC_GRID, 1, TC_BLOCK)
    res = _tc_loss(
        x_t, g, w64_t, tgt3,
        jnp.asarray(_NEG_PACKED), jnp.asarray(_REPL_PACKED),
    )
    return res.reshape(())


# speedup vs baseline: 7.6883x; 1.1028x over previous
import functools

import jax
import jax.numpy as jnp
import numpy as np
from jax import lax
from jax.experimental import pallas as pl
from jax.experimental.pallas import tpu as pltpu
from jax.experimental.pallas import tpu_sc as plsc

BATCH = 16384
DIM = 64
VOCAB = 100000
NOISE_VOCAB = 64
NUM_NEG = 5

NUM_CORES = 2
NUM_SUBCORES = 16
NUM_WORKERS = NUM_CORES * NUM_SUBCORES
D_PER_W = DIM // NUM_WORKERS
E_CHUNK = 4096
N_ECHUNK = BATCH // E_CHUNK
LANES = 16
UNROLL = 8

TC_BLOCK = 512
TC_GRID = BATCH // TC_BLOCK


def _threefry2x32(k1, k2, x0, x1):
    k1 = np.uint32(k1)
    k2 = np.uint32(k2)
    ks = (k1, k2, k1 ^ k2 ^ np.uint32(0x1BD11BDA))
    x0 = (x0 + ks[0]).astype(np.uint32)
    x1 = (x1 + ks[1]).astype(np.uint32)

    def rounds(x0, x1, rots):
        for r in rots:
            x0 = (x0 + x1).astype(np.uint32)
            x1 = (x1 << np.uint32(r)) | (x1 >> np.uint32(32 - r))
            x1 = x0 ^ x1
        return x0, x1

    rot_a, rot_b = (13, 15, 26, 6), (17, 29, 16, 24)
    inject = ((ks[1], ks[2], 1), (ks[2], ks[0], 2), (ks[0], ks[1], 3),
              (ks[1], ks[2], 4), (ks[2], ks[0], 5))
    for (a, b, c), rt in zip(inject, (rot_a, rot_b, rot_a, rot_b, rot_a)):
        x0, x1 = rounds(x0, x1, rt)
        x0 = (x0 + a).astype(np.uint32)
        x1 = (x1 + b + np.uint32(c)).astype(np.uint32)
    return x0, x1


def _iota_pair(size):
    n = np.arange(size, dtype=np.uint64)
    return ((n >> np.uint64(32)).astype(np.uint32),
            (n & np.uint64(0xFFFFFFFF)).astype(np.uint32))


def _categorical_uniform(k1, k2):
    hi, lo = _iota_pair(BATCH * NUM_NEG * NOISE_VOCAB)
    b1, b2 = _threefry2x32(k1, k2, hi, lo)
    bits = b1 ^ b2
    fb = (bits >> np.uint32(9)) | np.uint32(0x3F800000)
    floats = fb.view(np.float32) - np.float32(1.0)
    tiny = np.float32(np.finfo(np.float32).tiny)
    u = np.maximum(tiny, floats * np.float32(np.float32(1.0) - tiny) + tiny)
    g = -np.log(-np.log(u))
    return np.argmax(g.reshape(BATCH, NUM_NEG, NOISE_VOCAB), axis=-1).astype(np.int32)


def _noise_constants():
    hi, lo = _iota_pair(2)
    b1, b2 = _threefry2x32(0, 42, hi, lo)
    neg = _categorical_uniform(b1[0], b2[0])
    repl = _categorical_uniform(b1[1], b2[1])

    def pack(a):
        p = np.zeros((BATCH,), dtype=np.int64)
        for k in range(NUM_NEG):
            p |= a[:, k].astype(np.int64) << (6 * k)
        return p.astype(np.int32).reshape(TC_GRID, 1, TC_BLOCK)

    return pack(neg), pack(repl)


_NEG_PACKED, _REPL_PACKED = _noise_constants()


def _sc_gather_body(wt_hbm, tgt_hbm, out_hbm, tgt_v, row_v, g_v0, g_v1, sem_out):
    wid = lax.axis_index("s") * NUM_CORES + lax.axis_index("c")
    pltpu.sync_copy(tgt_hbm, tgt_v)
    pending = {}
    step = LANES * UNROLL
    for rep in range(D_PER_W):
        d = wid * D_PER_W + rep
        pltpu.sync_copy(wt_hbm.at[d], row_v)

        for c in range(N_ECHUNK):
            slot = c % 2
            if slot in pending:
                pending.pop(slot).wait()
            g_slot = (g_v0, g_v1)[slot]

            @plsc.parallel_loop(c * E_CHUNK, (c + 1) * E_CHUNK,
                                step=LANES, unroll=UNROLL)
            def body(i, c=c, g_slot=g_slot):
                idx16 = tgt_v[pl.ds(i, LANES)]
                g_slot[pl.ds(i - c * E_CHUNK, LANES)] = (
                    plsc.load_gather(row_v, [idx16]))
            pending[slot] = pltpu.async_copy(
                g_slot, out_hbm.at[d, pl.ds(c * E_CHUNK, E_CHUNK)],
                sem_out.at[slot])
    for cp in pending.values():
        cp.wait()


@functools.lru_cache(maxsize=None)
def _build_sc_gather():
    return functools.partial(
        pl.kernel,
        mesh=plsc.VectorSubcoreMesh(
            core_axis_name="c", subcore_axis_name="s",
            num_cores=NUM_CORES, num_subcores=NUM_SUBCORES,
        ),
        out_type=jax.ShapeDtypeStruct((DIM, BATCH), jnp.float32),
        compiler_params=pltpu.CompilerParams(
            use_tc_tiling_on_sc=True, needs_layout_passes=False),
        scratch_types=[
            pltpu.VMEM((BATCH,), jnp.int32),
            pltpu.VMEM((VOCAB,), jnp.float32),
            pltpu.VMEM((E_CHUNK,), jnp.float32),
            pltpu.VMEM((E_CHUNK,), jnp.float32),
            pltpu.SemaphoreType.DMA((2,)),
        ],
    )(_sc_gather_body)


def _log_sigmoid(z):
    return jnp.minimum(z, 0.0) - jnp.log1p(jnp.exp(-jnp.abs(z)))


def _tc_loss_body(x_ref, g_ref, w_ref, tgt_ref, negp_ref, replp_ref, out_ref):
    i = pl.program_id(0)

    x = x_ref[...]
    g = g_ref[...]
    w = w_ref[...]
    tgt = tgt_ref[0]
    negp = negp_ref[0]
    replp = replp_ref[0]

    s_t = lax.dot_general(
        w, x, dimension_numbers=(((0,), (0,)), ((), ())),
        preferred_element_type=jnp.float32,
    )

    iota_v = lax.broadcasted_iota(jnp.int32, (NOISE_VOCAB, TC_BLOCK), 0)
    total = jnp.float32(0.0)
    for k in range(NUM_NEG):
        nk = (negp >> (6 * k)) & 63
        rk = (replp >> (6 * k)) & 63
        nwk = jnp.where(nk == tgt, rk, nk)
        sel = jnp.where(iota_v == nwk, s_t, 0.0)
        sk = jnp.sum(sel, axis=0)
        total += jnp.sum(_log_sigmoid(-sk))

    pos_score = jnp.sum(x * g, axis=0)
    total += jnp.sum(_log_sigmoid(pos_score))

    @pl.when(i == 0)
    def _init():
        out_ref[...] = jnp.zeros((1, 1), jnp.float32)

    out_ref[...] += jnp.full((1, 1), total, jnp.float32)

    @pl.when(i == TC_GRID - 1)
    def _fin():
        out_ref[...] = out_ref[...] * (-1.0 / BATCH)


_tc_loss = pl.pallas_call(
    _tc_loss_body,
    grid=(TC_GRID,),
    in_specs=[
        pl.BlockSpec((DIM, TC_BLOCK), lambda i: (0, i)),
        pl.BlockSpec((DIM, TC_BLOCK), lambda i: (0, i)),
        pl.BlockSpec((DIM, NOISE_VOCAB), lambda i: (0, 0)),
        pl.BlockSpec((1, 1, TC_BLOCK), lambda i: (i, 0, 0)),
        pl.BlockSpec((1, 1, TC_BLOCK), lambda i: (i, 0, 0)),
        pl.BlockSpec((1, 1, TC_BLOCK), lambda i: (i, 0, 0)),
    ],
    out_specs=pl.BlockSpec((1, 1), lambda i: (0, 0)),
    out_shape=jax.ShapeDtypeStruct((1, 1), jnp.float32),
)


def kernel(input_embeddings, target_words, out_emb_weight):
    w_t = out_emb_weight.T
    x_t = input_embeddings.T
    g = _build_sc_gather()(w_t, target_words)
    w64_t = lax.slice(w_t, (0, 0), (DIM, NOISE_VOCAB))
    tgt3 = target_words.reshape(TC_GRID, 1, TC_BLOCK)
    res = _tc_loss(
        x_t, g, w64_t, tgt3,
        jnp.asarray(_NEG_PACKED), jnp.asarray(_REPL_PACKED),
    )
    return res.reshape(())
